# baseline (speedup 1.0000x reference)
import functools

import jax
import jax.numpy as jnp
from jax import lax
from jax.experimental import pallas as pl
from jax.experimental.pallas import tpu as pltpu
from jax.experimental.pallas import tpu_sc as plsc

NUM_K = 1024
DIM = 64
PIX_BLOCK = 8


def _dist_body(x_ref, w_ref, inds_ref, loss_ref):
    x = x_ref[...]
    w = w_ref[...]
    diff = x[:, None, :] - w[None, :, :]
    d2 = jnp.sum(diff * diff, axis=-1)
    dist = d2 * lax.rsqrt(d2)
    m = jnp.min(dist, axis=-1, keepdims=True)
    eq = dist == m
    iota_k = lax.broadcasted_iota(jnp.int32, (PIX_BLOCK, NUM_K), 1)
    inds = jnp.min(jnp.where(eq, iota_k, NUM_K), axis=-1)
    inds_ref[0, 0, :] = inds
    d2_sel = jnp.min(jnp.where(eq, d2, jnp.inf), axis=-1)

    @pl.when(pl.program_id(0) == 0)
    def _():
        loss_ref[0, 0] = 0.0

    loss_ref[0, 0] += jnp.sum(d2_sel)


GATHER_W = 128


def _make_sc_gather(n_rows: int):
    info = plsc.get_sparse_core_info()
    nc, ns = info.num_cores, info.num_subcores
    nw = nc * ns
    b_per_w = n_rows // nw
    mesh = plsc.VectorSubcoreMesh(core_axis_name="c", subcore_axis_name="s")

    @functools.partial(
        pl.kernel,
        mesh=mesh,
        out_type=jax.ShapeDtypeStruct((n_rows, GATHER_W), jnp.float32),
        scratch_types=[
            pltpu.VMEM((b_per_w,), jnp.int32),
            pltpu.VMEM((b_per_w, GATHER_W), jnp.float32),
            pltpu.SemaphoreType.DMA,
        ],
    )
    def gather_k(table_hbm, idx_hbm, out_hbm, idx_v, rows_v, sem):
        wid = lax.axis_index("s") * nc + lax.axis_index("c")
        base = wid * b_per_w
        pltpu.sync_copy(idx_hbm.at[pl.ds(base, b_per_w)], idx_v)
        pltpu.async_copy(table_hbm.at[idx_v], rows_v, sem).wait()
        pltpu.sync_copy(rows_v, out_hbm.at[pl.ds(base, b_per_w)])

    return gather_k


def kernel(latents, weight):
    b, d, h, w_sp = latents.shape
    n = b * h * w_sp
    x = jnp.moveaxis(latents, 1, -1).reshape(n, d)
    nblk = n // PIX_BLOCK

    inds3, loss_sum = pl.pallas_call(
        _dist_body,
        grid=(nblk,),
        in_specs=[
            pl.BlockSpec((PIX_BLOCK, DIM), lambda i: (i, 0)),
            pl.BlockSpec((NUM_K, DIM), lambda i: (0, 0)),
        ],
        out_specs=[
            pl.BlockSpec((1, 1, PIX_BLOCK), lambda i: (i, 0, 0)),
            pl.BlockSpec(memory_space=pltpu.SMEM, block_shape=(1, 1), index_map=lambda i: (0, 0)),
        ],
        out_shape=[
            jax.ShapeDtypeStruct((nblk, 1, PIX_BLOCK), jnp.int32),
            jax.ShapeDtypeStruct((1, 1), jnp.float32),
        ],
        compiler_params=pltpu.CompilerParams(
            dimension_semantics=("arbitrary",),
        ),
    )(x, weight)

    inds = inds3.reshape(n)
    wpad = jnp.pad(weight, ((0, 0), (0, GATHER_W - d)))
    q = _make_sc_gather(n)(wpad, inds)[:, :d]
    quantized = jnp.moveaxis(q.reshape(b, h, w_sp, d), -1, 1)
    vq_loss = loss_sum[0, 0] * (1.25 / (n * d))
    return (vq_loss, quantized)


# speedup vs baseline: 1.1177x; 1.1177x over previous
import functools

import jax
import jax.numpy as jnp
from jax import lax
from jax.experimental import pallas as pl
from jax.experimental.pallas import tpu as pltpu
from jax.experimental.pallas import tpu_sc as plsc

NUM_K = 1024
DIM = 64
PIX_BLOCK = 32


def _dist_body(x_ref, w_ref, inds_ref, loss_ref):
    x = x_ref[...]
    w = w_ref[...]
    diff = x[:, None, :] - w[None, :, :]
    d2 = jnp.sum(diff * diff, axis=-1)
    dist = d2 * lax.rsqrt(d2)
    m = jnp.min(dist, axis=-1, keepdims=True)
    eq = dist == m
    iota_k = lax.broadcasted_iota(jnp.int32, (PIX_BLOCK, NUM_K), 1)
    inds = jnp.min(jnp.where(eq, iota_k, NUM_K), axis=-1)
    inds_ref[0, 0, :] = inds
    d2_sel = jnp.min(jnp.where(eq, d2, jnp.inf), axis=-1)

    @pl.when(pl.program_id(0) == 0)
    def _():
        loss_ref[0, 0] = 0.0

    loss_ref[0, 0] += jnp.sum(d2_sel)


GATHER_W = 128


def _make_sc_gather(n_rows: int):
    info = plsc.get_sparse_core_info()
    nc, ns = info.num_cores, info.num_subcores
    nw = nc * ns
    b_per_w = n_rows // nw
    mesh = plsc.VectorSubcoreMesh(core_axis_name="c", subcore_axis_name="s")

    @functools.partial(
        pl.kernel,
        mesh=mesh,
        out_type=jax.ShapeDtypeStruct((n_rows, GATHER_W), jnp.float32),
        scratch_types=[
            pltpu.VMEM((b_per_w,), jnp.int32),
            pltpu.VMEM((b_per_w, GATHER_W), jnp.float32),
            pltpu.SemaphoreType.DMA,
        ],
    )
    def gather_k(table_hbm, idx_hbm, out_hbm, idx_v, rows_v, sem):
        wid = lax.axis_index("s") * nc + lax.axis_index("c")
        base = wid * b_per_w
        pltpu.sync_copy(idx_hbm.at[pl.ds(base, b_per_w)], idx_v)
        pltpu.async_copy(table_hbm.at[idx_v], rows_v, sem).wait()
        pltpu.sync_copy(rows_v, out_hbm.at[pl.ds(base, b_per_w)])

    return gather_k


def kernel(latents, weight):
    b, d, h, w_sp = latents.shape
    n = b * h * w_sp
    x = jnp.moveaxis(latents, 1, -1).reshape(n, d)
    nblk = n // PIX_BLOCK

    inds3, loss_sum = pl.pallas_call(
        _dist_body,
        grid=(nblk,),
        in_specs=[
            pl.BlockSpec((PIX_BLOCK, DIM), lambda i: (i, 0)),
            pl.BlockSpec((NUM_K, DIM), lambda i: (0, 0)),
        ],
        out_specs=[
            pl.BlockSpec((1, 1, PIX_BLOCK), lambda i: (i, 0, 0)),
            pl.BlockSpec(memory_space=pltpu.SMEM, block_shape=(1, 1), index_map=lambda i: (0, 0)),
        ],
        out_shape=[
            jax.ShapeDtypeStruct((nblk, 1, PIX_BLOCK), jnp.int32),
            jax.ShapeDtypeStruct((1, 1), jnp.float32),
        ],
        compiler_params=pltpu.CompilerParams(
            dimension_semantics=("arbitrary",),
        ),
    )(x, weight)

    inds = inds3.reshape(n)
    wpad = jnp.pad(weight, ((0, 0), (0, GATHER_W - d)))
    q = _make_sc_gather(n)(wpad, inds)[:, :d]
    quantized = jnp.moveaxis(q.reshape(b, h, w_sp, d), -1, 1)
    vq_loss = loss_sum[0, 0] * (1.25 / (n * d))
    return (vq_loss, quantized)
